# baseline (speedup 1.0000x reference)
import functools

import jax
import jax.numpy as jnp
from jax import lax
from jax.experimental import pallas as pl
from jax.experimental.pallas import tpu as pltpu
from jax.experimental.pallas import tpu_sc as plsc

N, D = 32768, 512
B, L = 16, 4096

NC, NS, LANES = 2, 16, 16
NW = NC * NS
RPW = N // NW
CH = 64
NCH = RPW // CH
HALF = N // NC
IDX_PER = (B * L) // NS
GROUPS = CH // LANES
DJ = D // LANES


def _fused_body(x_hbm, w_hbm, bagsT_hbm, out_hbm,
                xbuf0, xbuf1, w_v, idx_v, accmat_v, mylog_v, halflog_v,
                part_v, half_sh, sem0, sem1):
    c = lax.axis_index("c")
    s = lax.axis_index("s")
    rowbase = (c * NS + s) * RPW

    pltpu.sync_copy(w_hbm, w_v)
    pltpu.sync_copy(bagsT_hbm.at[pl.ds(s * IDX_PER, IDX_PER)], idx_v)

    xb = (xbuf0, xbuf1)
    sems = (sem0, sem1)
    d0 = pltpu.async_copy(x_hbm.at[pl.ds(rowbase, CH), :], xbuf0, sem0)
    d1 = pltpu.async_copy(x_hbm.at[pl.ds(rowbase + CH, CH), :], xbuf1, sem1)
    del d0, d1

    wregs = [w_v[pl.ds(j * LANES, LANES)] for j in range(DJ)]

    def make_group(buf, k):
        def group(g, carry):
            for i in range(LANES):
                row = g * LANES + i
                acc = xb[buf][row, pl.ds(0, LANES)] * wregs[0]
                for j in range(1, DJ):
                    acc = acc + xb[buf][row, pl.ds(j * LANES, LANES)] * wregs[j]
                accmat_v[pl.ds(i * LANES, LANES)] = acc
            lanes = lax.iota(jnp.int32, LANES) * LANES
            tot = plsc.load_gather(accmat_v, [lanes])
            for t in range(1, LANES):
                tot = tot + plsc.load_gather(accmat_v, [lanes + t])
            mylog_v[pl.ds(k * CH + g * LANES, LANES)] = tot
            return carry
        return group

    def chunk_iter(it, carry):
        for bpar in range(2):
            k = it * 2 + bpar
            pltpu.make_async_copy(
                x_hbm.at[pl.ds(rowbase, CH), :], xb[bpar], sems[bpar]).wait()
            lax.fori_loop(0, GROUPS, make_group(bpar, k), 0)

            @pl.when(k + 2 < NCH)
            def _():
                pltpu.async_copy(
                    x_hbm.at[pl.ds(rowbase + (k + 2) * CH, CH), :],
                    xb[bpar], sems[bpar])
        return carry

    lax.fori_loop(0, NCH // 2, chunk_iter, 0)

    pltpu.sync_copy(mylog_v, half_sh.at[pl.ds(s * RPW, RPW)])
    plsc.subcore_barrier()
    pltpu.sync_copy(half_sh, halflog_v)

    base = c * HALF

    def gbody(j, acc):
        idx = idx_v[pl.ds(j * LANES, LANES)]
        local = idx - base
        valid = (local >= 0) & (local < HALF)
        clamped = jnp.minimum(jnp.maximum(local, 0), HALF - 1)
        vals = plsc.load_gather(halflog_v, [clamped])
        vals = jnp.where(valid, vals, -jnp.inf)
        return jnp.maximum(acc, vals)

    acc = lax.fori_loop(0, IDX_PER // LANES, gbody,
                        jnp.full((LANES,), -jnp.inf, jnp.float32))

    part_v[...] = acc
    pltpu.sync_copy(part_v, out_hbm.at[c * NS + s])


_fused = functools.partial(
    pl.kernel,
    out_type=jax.ShapeDtypeStruct((NW, LANES), jnp.float32),
    mesh=plsc.VectorSubcoreMesh(
        core_axis_name="c", subcore_axis_name="s",
        num_cores=NC, num_subcores=NS),
    compiler_params=pltpu.CompilerParams(needs_layout_passes=False),
    scratch_types=[
        pltpu.VMEM((CH, D), jnp.float32),
        pltpu.VMEM((CH, D), jnp.float32),
        pltpu.VMEM((D,), jnp.float32),
        pltpu.VMEM((IDX_PER,), jnp.int32),
        pltpu.VMEM((LANES * LANES,), jnp.float32),
        pltpu.VMEM((RPW,), jnp.float32),
        pltpu.VMEM((HALF,), jnp.float32),
        pltpu.VMEM((LANES,), jnp.float32),
        pltpu.VMEM_SHARED((HALF,), jnp.float32),
        pltpu.SemaphoreType.DMA,
        pltpu.SemaphoreType.DMA,
    ],
)(_fused_body)


def kernel(X, bags, bags_mask, W, b):
    bagsT = bags.T.reshape(L * B)
    part = _fused(X, W.reshape(D), bagsT)
    m = (jnp.max(part, axis=0) + b[0]).reshape(B, 1)
    p = jax.nn.sigmoid(m)
    return jnp.log(jnp.concatenate([1.0 - p, p], axis=1))


# speedup vs baseline: 1.2472x; 1.2472x over previous
import functools

import jax
import jax.numpy as jnp
from jax import lax
from jax.experimental import pallas as pl
from jax.experimental.pallas import tpu as pltpu
from jax.experimental.pallas import tpu_sc as plsc

N, D = 32768, 512
B, L = 16, 4096

NC, NS, LANES = 2, 16, 16
NW = NC * NS
RPW = N // NW
CH = 64
NCH = RPW // CH
HALF = N // NC
IDX_PER = (B * L) // NS
GROUPS = CH // LANES
DJ = D // LANES


def _fused_body(x_hbm, w_hbm, bagsT_hbm, out_hbm,
                xbuf0, xbuf1, w_v, idx_v, accmat_v, mylog_v, halflog_v,
                part_v, half_sh, sem0, sem1):
    c = lax.axis_index("c")
    s = lax.axis_index("s")
    rowbase = (c * NS + s) * RPW

    pltpu.sync_copy(w_hbm, w_v)
    pltpu.sync_copy(bagsT_hbm.at[pl.ds(s * IDX_PER, IDX_PER)], idx_v)

    xb = (xbuf0, xbuf1)
    sems = (sem0, sem1)
    d0 = pltpu.async_copy(x_hbm.at[pl.ds(rowbase, CH), :], xbuf0, sem0)
    d1 = pltpu.async_copy(x_hbm.at[pl.ds(rowbase + CH, CH), :], xbuf1, sem1)
    del d0, d1

    wregs = [w_v[pl.ds(j * LANES, LANES)] for j in range(DJ)]

    def make_group(buf, k):
        def group(g, carry):
            for i in range(LANES):
                row = g * LANES + i
                prods = [xb[buf][row, pl.ds(j * LANES, LANES)] * wregs[j]
                         for j in range(DJ)]
                while len(prods) > 1:
                    prods = [prods[m] + prods[m + 1]
                             for m in range(0, len(prods), 2)]
                accmat_v[pl.ds(i * LANES, LANES)] = prods[0]
            lanes = lax.iota(jnp.int32, LANES) * LANES
            gath = [plsc.load_gather(accmat_v, [lanes + t]) for t in range(LANES)]
            while len(gath) > 1:
                gath = [gath[m] + gath[m + 1] for m in range(0, len(gath), 2)]
            mylog_v[pl.ds(k * CH + g * LANES, LANES)] = gath[0]
            return carry
        return group

    def chunk_iter(it, carry):
        for bpar in range(2):
            k = it * 2 + bpar
            pltpu.make_async_copy(
                x_hbm.at[pl.ds(rowbase, CH), :], xb[bpar], sems[bpar]).wait()
            lax.fori_loop(0, GROUPS, make_group(bpar, k), 0)

            @pl.when(k + 2 < NCH)
            def _():
                pltpu.async_copy(
                    x_hbm.at[pl.ds(rowbase + (k + 2) * CH, CH), :],
                    xb[bpar], sems[bpar])
        return carry

    lax.fori_loop(0, NCH // 2, chunk_iter, 0)

    pltpu.sync_copy(mylog_v, half_sh.at[pl.ds(s * RPW, RPW)])
    plsc.subcore_barrier()
    pltpu.sync_copy(half_sh, halflog_v)

    base = c * HALF

    def gbody(j, acc):
        idx = idx_v[pl.ds(j * LANES, LANES)]
        local = idx - base
        valid = (local >= 0) & (local < HALF)
        clamped = jnp.minimum(jnp.maximum(local, 0), HALF - 1)
        vals = plsc.load_gather(halflog_v, [clamped])
        vals = jnp.where(valid, vals, -jnp.inf)
        return jnp.maximum(acc, vals)

    acc = lax.fori_loop(0, IDX_PER // LANES, gbody,
                        jnp.full((LANES,), -jnp.inf, jnp.float32))

    part_v[...] = acc
    pltpu.sync_copy(part_v, out_hbm.at[c * NS + s])


_fused = functools.partial(
    pl.kernel,
    out_type=jax.ShapeDtypeStruct((NW, LANES), jnp.float32),
    mesh=plsc.VectorSubcoreMesh(
        core_axis_name="c", subcore_axis_name="s",
        num_cores=NC, num_subcores=NS),
    compiler_params=pltpu.CompilerParams(needs_layout_passes=False),
    scratch_types=[
        pltpu.VMEM((CH, D), jnp.float32),
        pltpu.VMEM((CH, D), jnp.float32),
        pltpu.VMEM((D,), jnp.float32),
        pltpu.VMEM((IDX_PER,), jnp.int32),
        pltpu.VMEM((LANES * LANES,), jnp.float32),
        pltpu.VMEM((RPW,), jnp.float32),
        pltpu.VMEM((HALF,), jnp.float32),
        pltpu.VMEM((LANES,), jnp.float32),
        pltpu.VMEM_SHARED((HALF,), jnp.float32),
        pltpu.SemaphoreType.DMA,
        pltpu.SemaphoreType.DMA,
    ],
)(_fused_body)


def kernel(X, bags, bags_mask, W, b):
    bagsT = bags.T.reshape(L * B)
    part = _fused(X, W.reshape(D), bagsT)
    m = (jnp.max(part, axis=0) + b[0]).reshape(B, 1)
    p = jax.nn.sigmoid(m)
    return jnp.log(jnp.concatenate([1.0 - p, p], axis=1))
